# baseline (speedup 1.0000x reference)
import functools

import jax
import jax.numpy as jnp
from jax import lax
from jax.experimental import pallas as pl
from jax.experimental.pallas import tpu as pltpu
from jax.experimental.pallas import tpu_sc as plsc

MAX_OUT = 100
IOU_THR = 0.5
SCORE_THR = 0.01
NPAD = 20480
NEG_INF = float("-inf")

L = 16
NCHUNK = NPAD // L
NM1V = NCHUNK // L
NM2V = NM1V // L
SELPAD = 112
PT = NPAD // 8
CK = 256
NCK = PT // CK
NG = CK // L


def _sc_all(lgp, dbt):
    mesh = plsc.VectorSubcoreMesh(core_axis_name="c", subcore_axis_name="s")

    @functools.partial(
        pl.kernel,
        mesh=mesh,
        out_type=[
            jax.ShapeDtypeStruct((4 * SELPAD,), jnp.float32),
            jax.ShapeDtypeStruct((4 * SELPAD,), jnp.float32),
            jax.ShapeDtypeStruct((4 * SELPAD,), jnp.float32),
            jax.ShapeDtypeStruct((4 * SELPAD,), jnp.float32),
            jax.ShapeDtypeStruct((4 * SELPAD,), jnp.int32),
            jax.ShapeDtypeStruct((4 * SELPAD,), jnp.float32),
            jax.ShapeDtypeStruct((4 * L,), jnp.int32),
            jax.ShapeDtypeStruct((4 * NPAD,), jnp.int32),
            jax.ShapeDtypeStruct((4 * 5 * NPAD,), jnp.float32),
        ],
        scratch_types=[
            pltpu.VMEM((NPAD,), jnp.float32),
            pltpu.VMEM((NPAD,), jnp.float32),
            pltpu.VMEM((NPAD,), jnp.float32),
            pltpu.VMEM((NPAD,), jnp.float32),
            pltpu.VMEM((NPAD,), jnp.float32),
            pltpu.VMEM((NCHUNK,), jnp.float32),
            pltpu.VMEM((NM1V,), jnp.float32),
            pltpu.VMEM((SELPAD,), jnp.float32),
            pltpu.VMEM((SELPAD,), jnp.float32),
            pltpu.VMEM((SELPAD,), jnp.float32),
            pltpu.VMEM((SELPAD,), jnp.float32),
            pltpu.VMEM((SELPAD,), jnp.float32),
            pltpu.VMEM((SELPAD,), jnp.int32),
            pltpu.VMEM((SELPAD,), jnp.float32),
            pltpu.VMEM((L,), jnp.int32),
            pltpu.VMEM((PT,), jnp.int32),
            pltpu.VMEM((CK * 25,), jnp.float32),
            pltpu.VMEM((CK * 25,), jnp.float32),
            pltpu.VMEM((CK * 4,), jnp.float32),
            pltpu.VMEM((CK * 4,), jnp.float32),
            pltpu.SemaphoreType.DMA,
            pltpu.SemaphoreType.DMA,
            pltpu.SemaphoreType.DMA,
        ],
        compiler_params=pltpu.CompilerParams(needs_layout_passes=False),
    )
    def k(lg_hbm, db_hbm, ox1_hbm, oy1_hbm, ox2_hbm, oy2_hbm, oi_hbm, os_hbm,
          on_hbm, co_hbm, fi_hbm,
          s_v, x1_v, y1_v, x2_v, y2_v, m1_v, m2_v,
          sx1_v, sy1_v, sx2_v, sy2_v, sar_v,
          oi_v, os_v, on_v, cl_v,
          rb0, rb1, db0, db1, sem0, sem1, semp):
        cc = lax.axis_index("c")
        ss = lax.axis_index("s")
        iota16 = lax.iota(jnp.int32, 16)
        zeros16 = jnp.zeros((L,), jnp.float32)
        izeros16 = jnp.zeros((L,), jnp.int32)

        img = 2 * cc + ss // 8
        part = ss % 8
        a0 = part * PT
        lbase = (img * NPAD + a0) * 25

        rbs = (rb0, rb1)
        dbs = (db0, db1)
        sems = (sem0, sem1)

        def fire(kk, slot):
            h = [
                pltpu.async_copy(
                    lg_hbm.at[pl.ds(lbase + kk * CK * 25, CK * 25)],
                    rbs[slot], sems[slot],
                )
            ]
            for comp in range(4):
                h.append(
                    pltpu.async_copy(
                        db_hbm.at[pl.ds(comp * NPAD + a0 + kk * CK, CK)],
                        dbs[slot].at[pl.ds(comp * CK, CK)], sems[slot],
                    )
                )
            return h

        pend = {0: fire(0, 0), 1: None}
        for kk in range(NCK):
            slot = kk % 2
            for h in pend[slot]:
                h.wait()
            if kk + 1 < NCK:
                pend[(kk + 1) % 2] = fire(kk + 1, (kk + 1) % 2)
            rb = rbs[slot]
            dbv = dbs[slot]

            def gbody(g, carry, rb=rb, dbv=dbv, kk=kk):
                ab = g * L
                l = []
                for ci in range(25):
                    l.append(
                        plsc.load_gather(rb, [(ab + iota16) * 25 + ci])
                    )
                ax1 = dbv[pl.ds(ab, L)]
                ay1 = dbv[pl.ds(CK + ab, L)]
                ax2 = dbv[pl.ds(2 * CK + ab, L)]
                ay2 = dbv[pl.ds(3 * CK + ab, L)]
                acx = (ax2 + ax1) * 0.5
                acy = (ay2 + ay1) * 0.5
                aw = ax2 - ax1
                ah = ay2 - ay1
                pcx = l[0] * aw + acx
                pcy = l[1] * ah + acy
                pw = jnp.exp(l[2]) * aw
                ph = jnp.exp(l[3]) * ah
                x1 = jnp.clip(pcx - pw * 0.5, 0.0, 1.0)
                y1 = jnp.clip(pcy - ph * 0.5, 0.0, 1.0)
                x2 = jnp.clip(pcx + pw * 0.5, 0.0, 1.0)
                y2 = jnp.clip(pcy + ph * 0.5, 0.0, 1.0)

                m = l[4]
                for ci in range(5, 25):
                    m = jnp.maximum(m, l[ci])
                ssum = jnp.exp(l[4] - m)
                best = l[4]
                cls = izeros16
                for ci in range(5, 25):
                    lc = l[ci]
                    ssum = ssum + jnp.exp(lc - m)
                    gt = lc > best
                    best = jnp.where(gt, lc, best)
                    cls = jnp.where(gt, jnp.int32(ci - 4), cls)
                score = 1.0 / ssum
                s0 = jnp.where(cls != 0, score, NEG_INF)
                s0 = jnp.where(s0 < SCORE_THR, NEG_INF, s0)

                off = kk * CK + ab
                s_v[pl.ds(off, L)] = s0
                x1_v[pl.ds(off, L)] = x1
                y1_v[pl.ds(off, L)] = y1
                x2_v[pl.ds(off, L)] = x2
                y2_v[pl.ds(off, L)] = y2
                cl_v[pl.ds(off, L)] = cls
                return carry

            lax.fori_loop(0, NG, gbody, jnp.int32(0))

        hs = []
        for plane, srcv in enumerate((s_v, x1_v, y1_v, x2_v, y2_v)):
            hs.append(
                pltpu.async_copy(
                    srcv.at[pl.ds(0, PT)],
                    fi_hbm.at[pl.ds((img * 5 + plane) * NPAD + a0, PT)],
                    semp,
                )
            )
        hs.append(
            pltpu.async_copy(
                cl_v.at[pl.ds(0, PT)],
                co_hbm.at[pl.ds(img * NPAD + a0, PT)], semp,
            )
        )
        for h in hs:
            h.wait()

        plsc.subcore_barrier()

        @pl.when(ss < 2)
        def _():
            wid = 2 * cc + ss
            hs2 = []
            for plane, dst in enumerate((s_v, x1_v, y1_v, x2_v, y2_v)):
                hs2.append(
                    pltpu.async_copy(
                        fi_hbm.at[pl.ds((wid * 5 + plane) * NPAD, NPAD)],
                        dst, semp,
                    )
                )
            for h in hs2:
                h.wait()

            for qq in range(SELPAD // L):
                oi_v[pl.ds(qq * L, L)] = izeros16
                os_v[pl.ds(qq * L, L)] = zeros16
                sx1_v[pl.ds(qq * L, L)] = zeros16
                sy1_v[pl.ds(qq * L, L)] = zeros16
                sx2_v[pl.ds(qq * L, L)] = zeros16
                sy2_v[pl.ds(qq * L, L)] = zeros16
                sar_v[pl.ds(qq * L, L)] = zeros16
            on_v[pl.ds(0, L)] = izeros16

            def m1_body(r, carry):
                w = jnp.full((L,), NEG_INF, jnp.float32)
                for j in range(L):
                    cj = jnp.max(s_v[pl.ds(r * 256 + j * L, L)], axis=0)
                    w = jnp.where(iota16 == j, cj, w)
                m1_v[pl.ds(r * L, L)] = w
                return carry

            lax.fori_loop(0, NM1V, m1_body, jnp.int32(0))

            for i in range(NM2V):
                w = jnp.full((L,), NEG_INF, jnp.float32)
                for j in range(L):
                    cj = jnp.max(m1_v[pl.ds(i * 256 + j * L, L)], axis=0)
                    w = jnp.where(iota16 == j, cj, w)
                m2_v[pl.ds(i * L, L)] = w

            big = jnp.int32(NPAD)

            def global_argmax():
                g = m2_v[pl.ds(0, L)]
                for qj in range(1, NM2V):
                    g = jnp.maximum(g, m2_v[pl.ds(qj * L, L)])
                gmax = jnp.max(g, axis=0)
                pos = jnp.full((L,), NPAD, jnp.int32)
                for qj in range(NM2V):
                    vv = m2_v[pl.ds(qj * L, L)]
                    pos = jnp.minimum(
                        pos, jnp.where(vv == gmax, qj * L + iota16, big)
                    )
                p1 = jnp.min(pos, axis=0)
                m1c = m1_v[pl.ds(p1 * L, L)]
                ch = jnp.min(
                    jnp.where(m1c == gmax, p1 * L + iota16, big), axis=0
                )
                sch = s_v[pl.ds(ch * L, L)]
                idx = jnp.min(
                    jnp.where(sch == gmax, ch * L + iota16, big), axis=0
                )
                return gmax, idx

            def extract_f(ref, idx, lane):
                v = ref[pl.ds(idx - lane, L)]
                return jnp.max(jnp.where(iota16 == lane, v, NEG_INF), axis=0)

            def append(ref, val, qn, ln):
                v = ref[pl.ds(qn * L, L)]
                ref[pl.ds(qn * L, L)] = jnp.where(iota16 == ln, val, v)

            def cond(state):
                n, gmax, _idx = state
                return (n < MAX_OUT) & (gmax != NEG_INF)

            def body(state):
                n, gmax, idx = state
                lane = idx % L
                ch = idx // L
                bx1 = extract_f(x1_v, idx, lane)
                by1 = extract_f(y1_v, idx, lane)
                bx2 = extract_f(x2_v, idx, lane)
                by2 = extract_f(y2_v, idx, lane)
                barea = (bx2 - bx1) * (by2 - by1)

                sup = jnp.zeros((L,), jnp.int32)
                for qj in range(SELPAD // L):
                    qx1 = sx1_v[pl.ds(qj * L, L)]
                    qy1 = sy1_v[pl.ds(qj * L, L)]
                    qx2 = sx2_v[pl.ds(qj * L, L)]
                    qy2 = sy2_v[pl.ds(qj * L, L)]
                    qar = sar_v[pl.ds(qj * L, L)]
                    xx1 = jnp.maximum(bx1, qx1)
                    yy1 = jnp.maximum(by1, qy1)
                    xx2 = jnp.minimum(bx2, qx2)
                    yy2 = jnp.minimum(by2, qy2)
                    inter = jnp.maximum(xx2 - xx1, 0.0) * jnp.maximum(
                        yy2 - yy1, 0.0
                    )
                    iou = inter / (barea + qar - inter + 1e-9)
                    hit = (qj * L + iota16 < n) & (iou > IOU_THR)
                    sup = sup | hit.astype(jnp.int32)
                keep = jnp.max(sup, axis=0) == 0

                sch = s_v[pl.ds(ch * L, L)]
                sch = jnp.where(iota16 == lane, NEG_INF, sch)
                s_v[pl.ds(ch * L, L)] = sch
                cmax = jnp.max(sch, axis=0)
                q1 = ch // L
                l1 = ch % L
                m1c = m1_v[pl.ds(q1 * L, L)]
                m1c = jnp.where(iota16 == l1, cmax, m1c)
                m1_v[pl.ds(q1 * L, L)] = m1c
                nm1 = jnp.max(m1c, axis=0)
                q2 = q1 // L
                l2 = q1 % L
                m2c = m2_v[pl.ds(q2 * L, L)]
                m2c = jnp.where(iota16 == l2, nm1, m2c)
                m2_v[pl.ds(q2 * L, L)] = m2c

                @pl.when(keep)
                def _():
                    qn = n // L
                    ln = n % L
                    append(sx1_v, bx1, qn, ln)
                    append(sy1_v, by1, qn, ln)
                    append(sx2_v, bx2, qn, ln)
                    append(sy2_v, by2, qn, ln)
                    append(sar_v, barea, qn, ln)
                    append(oi_v, idx, qn, ln)
                    append(os_v, gmax, qn, ln)

                n = n + keep.astype(jnp.int32)
                gmax, idx = global_argmax()
                return n, gmax, idx

            gmax0, idx0 = global_argmax()
            state = lax.while_loop(cond, body, (jnp.int32(0), gmax0, idx0))
            nfin = state[0]
            onv = on_v[pl.ds(0, L)]
            on_v[pl.ds(0, L)] = jnp.where(iota16 == 0, nfin, onv)

            pltpu.sync_copy(sx1_v, ox1_hbm.at[pl.ds(wid * SELPAD, SELPAD)])
            pltpu.sync_copy(sy1_v, oy1_hbm.at[pl.ds(wid * SELPAD, SELPAD)])
            pltpu.sync_copy(sx2_v, ox2_hbm.at[pl.ds(wid * SELPAD, SELPAD)])
            pltpu.sync_copy(sy2_v, oy2_hbm.at[pl.ds(wid * SELPAD, SELPAD)])
            pltpu.sync_copy(oi_v, oi_hbm.at[pl.ds(wid * SELPAD, SELPAD)])
            pltpu.sync_copy(os_v, os_hbm.at[pl.ds(wid * SELPAD, SELPAD)])
            pltpu.sync_copy(on_v, on_hbm.at[pl.ds(wid * L, L)])

    return k(lgp, dbt)


@jax.jit
def _run(lgp, dbt):
    ox1, oy1, ox2, oy2, oi, os_, on, co, _fi = _sc_all(lgp, dbt)
    nvec = on.reshape(4, L)[:, :1]
    mask = jnp.arange(MAX_OUT)[None, :] < nvec
    det_boxes = jnp.stack(
        [
            jnp.where(mask, ox1.reshape(4, SELPAD)[:, :MAX_OUT], 0.0),
            jnp.where(mask, oy1.reshape(4, SELPAD)[:, :MAX_OUT], 0.0),
            jnp.where(mask, ox2.reshape(4, SELPAD)[:, :MAX_OUT], 0.0),
            jnp.where(mask, oy2.reshape(4, SELPAD)[:, :MAX_OUT], 0.0),
        ],
        axis=-1,
    )
    idxs = oi.reshape(4, SELPAD)[:, :MAX_OUT]
    det_cls = jnp.take_along_axis(co.reshape(4, NPAD), idxs, axis=1)
    det_cls = jnp.where(mask, det_cls, 0)
    return (
        det_boxes,
        det_cls,
        os_.reshape(4, SELPAD)[:, :MAX_OUT],
        on.reshape(4, L)[:, 0],
    )


def kernel(logits, default_boxes):
    b, n, c = logits.shape
    lgp = jnp.pad(logits, ((0, 0), (0, NPAD - n), (0, 0))).reshape(
        b * NPAD * c
    )
    dbt = jnp.pad(default_boxes.T, ((0, 0), (0, NPAD - n))).reshape(4 * NPAD)
    return _run(lgp, dbt)


# speedup vs baseline: 2.0587x; 2.0587x over previous
import functools

import jax
import jax.numpy as jnp
from jax import lax
from jax.experimental import pallas as pl
from jax.experimental.pallas import tpu as pltpu
from jax.experimental.pallas import tpu_sc as plsc

MAX_OUT = 100
IOU_THR = 0.5
SCORE_THR = 0.01
ROWS = 160
LANES = 128
NPAD = ROWS * LANES
NEG_INF = float("-inf")

L = 16
NCHUNK = NPAD // L
NM1V = NCHUNK // L
NM2V = NM1V // L
SELPAD = 112


def _prep_kernel(lt_ref, db_ref, f_ref, c_ref):
    ax1 = db_ref[0]
    ay1 = db_ref[1]
    ax2 = db_ref[2]
    ay2 = db_ref[3]
    acx = (ax2 + ax1) * 0.5
    acy = (ay2 + ay1) * 0.5
    aw = ax2 - ax1
    ah = ay2 - ay1

    pcx = lt_ref[0, 0] * aw + acx
    pcy = lt_ref[0, 1] * ah + acy
    pw = jnp.exp(lt_ref[0, 2]) * aw
    ph = jnp.exp(lt_ref[0, 3]) * ah
    f_ref[0, 1] = jnp.clip(pcx - pw * 0.5, 0.0, 1.0)
    f_ref[0, 2] = jnp.clip(pcy - ph * 0.5, 0.0, 1.0)
    f_ref[0, 3] = jnp.clip(pcx + pw * 0.5, 0.0, 1.0)
    f_ref[0, 4] = jnp.clip(pcy + ph * 0.5, 0.0, 1.0)

    m = lt_ref[0, 4]
    for c in range(5, 25):
        m = jnp.maximum(m, lt_ref[0, c])
    ssum = jnp.exp(lt_ref[0, 4] - m)
    best = lt_ref[0, 4]
    cls = jnp.zeros((ROWS, LANES), dtype=jnp.int32)
    for c in range(5, 25):
        lc = lt_ref[0, c]
        ssum = ssum + jnp.exp(lc - m)
        gt = lc > best
        best = jnp.where(gt, lc, best)
        cls = jnp.where(gt, jnp.int32(c - 4), cls)
    score = 1.0 / ssum
    c_ref[0] = cls
    s0 = jnp.where(cls != 0, score, NEG_INF)
    s0 = jnp.where(s0 < SCORE_THR, NEG_INF, s0)
    f_ref[0, 0] = s0


@jax.jit
def _prep(lt, db):
    return pl.pallas_call(
        _prep_kernel,
        grid=(4,),
        in_specs=[
            pl.BlockSpec((1, 25, ROWS, LANES), lambda b: (b, 0, 0, 0)),
            pl.BlockSpec((4, ROWS, LANES), lambda b: (0, 0, 0)),
        ],
        out_specs=[
            pl.BlockSpec((1, 5, ROWS, LANES), lambda b: (b, 0, 0, 0)),
            pl.BlockSpec((1, ROWS, LANES), lambda b: (b, 0, 0)),
        ],
        out_shape=[
            jax.ShapeDtypeStruct((4, 5, ROWS, LANES), jnp.float32),
            jax.ShapeDtypeStruct((4, ROWS, LANES), jnp.int32),
        ],
        compiler_params=pltpu.CompilerParams(
            dimension_semantics=("arbitrary",),
        ),
    )(lt, db)


def _sc_nms(f2, c2):
    mesh = plsc.VectorSubcoreMesh(core_axis_name="c", subcore_axis_name="s")

    @functools.partial(
        pl.kernel,
        mesh=mesh,
        out_type=[
            jax.ShapeDtypeStruct((4 * SELPAD,), jnp.float32),
            jax.ShapeDtypeStruct((4 * SELPAD,), jnp.float32),
            jax.ShapeDtypeStruct((4 * SELPAD,), jnp.float32),
            jax.ShapeDtypeStruct((4 * SELPAD,), jnp.float32),
            jax.ShapeDtypeStruct((4 * SELPAD,), jnp.int32),
            jax.ShapeDtypeStruct((4 * SELPAD,), jnp.float32),
            jax.ShapeDtypeStruct((4 * L,), jnp.int32),
        ],
        scratch_types=[
            pltpu.VMEM((NPAD,), jnp.float32),
            pltpu.VMEM((NPAD,), jnp.float32),
            pltpu.VMEM((NPAD,), jnp.float32),
            pltpu.VMEM((NPAD,), jnp.float32),
            pltpu.VMEM((NPAD,), jnp.float32),
            pltpu.VMEM((NPAD,), jnp.int32),
            pltpu.VMEM((NCHUNK,), jnp.float32),
            pltpu.VMEM((NM1V,), jnp.float32),
            pltpu.VMEM((SELPAD,), jnp.float32),
            pltpu.VMEM((SELPAD,), jnp.float32),
            pltpu.VMEM((SELPAD,), jnp.float32),
            pltpu.VMEM((SELPAD,), jnp.float32),
            pltpu.VMEM((SELPAD,), jnp.float32),
            pltpu.VMEM((SELPAD,), jnp.int32),
            pltpu.VMEM((SELPAD,), jnp.float32),
            pltpu.VMEM((L,), jnp.int32),
        ],
        compiler_params=pltpu.CompilerParams(needs_layout_passes=False),
    )
    def k(f_hbm, c_hbm, ox1_hbm, oy1_hbm, ox2_hbm, oy2_hbm, oc_hbm, os_hbm,
          on_hbm,
          s_v, x1_v, y1_v, x2_v, y2_v, cl_v, m1_v, m2_v,
          sx1_v, sy1_v, sx2_v, sy2_v, sar_v,
          oc_v, os_v, on_v):
        wid = lax.axis_index("c") * 16 + lax.axis_index("s")

        @pl.when(wid < 4)
        def _():
            iota16 = lax.iota(jnp.int32, 16)
            zeros16 = jnp.zeros((L,), jnp.float32)
            izeros16 = jnp.zeros((L,), jnp.int32)

            base = wid * 5 * NPAD
            pltpu.sync_copy(f_hbm.at[pl.ds(base, NPAD)], s_v)
            pltpu.sync_copy(f_hbm.at[pl.ds(base + NPAD, NPAD)], x1_v)
            pltpu.sync_copy(f_hbm.at[pl.ds(base + 2 * NPAD, NPAD)], y1_v)
            pltpu.sync_copy(f_hbm.at[pl.ds(base + 3 * NPAD, NPAD)], x2_v)
            pltpu.sync_copy(f_hbm.at[pl.ds(base + 4 * NPAD, NPAD)], y2_v)
            pltpu.sync_copy(c_hbm.at[pl.ds(wid * NPAD, NPAD)], cl_v)

            for q in range(SELPAD // L):
                oc_v[pl.ds(q * L, L)] = izeros16
                os_v[pl.ds(q * L, L)] = zeros16
                sx1_v[pl.ds(q * L, L)] = zeros16
                sy1_v[pl.ds(q * L, L)] = zeros16
                sx2_v[pl.ds(q * L, L)] = zeros16
                sy2_v[pl.ds(q * L, L)] = zeros16
                sar_v[pl.ds(q * L, L)] = zeros16
            on_v[pl.ds(0, L)] = izeros16

            def m1_body(r, carry):
                w = jnp.full((L,), NEG_INF, jnp.float32)
                for j in range(L):
                    cj = jnp.max(s_v[pl.ds(r * 256 + j * L, L)], axis=0)
                    w = jnp.where(iota16 == j, cj, w)
                m1_v[pl.ds(r * L, L)] = w
                return carry

            lax.fori_loop(0, NM1V, m1_body, jnp.int32(0))

            for i in range(NM2V):
                w = jnp.full((L,), NEG_INF, jnp.float32)
                for j in range(L):
                    cj = jnp.max(m1_v[pl.ds(i * 256 + j * L, L)], axis=0)
                    w = jnp.where(iota16 == j, cj, w)
                m2_v[pl.ds(i * L, L)] = w

            big = jnp.int32(NPAD)

            def global_argmax():
                g = m2_v[pl.ds(0, L)]
                for q in range(1, NM2V):
                    g = jnp.maximum(g, m2_v[pl.ds(q * L, L)])
                gmax = jnp.max(g, axis=0)
                pos = jnp.full((L,), NPAD, jnp.int32)
                for q in range(NM2V):
                    vv = m2_v[pl.ds(q * L, L)]
                    pos = jnp.minimum(
                        pos, jnp.where(vv == gmax, q * L + iota16, big)
                    )
                p1 = jnp.min(pos, axis=0)
                m1c = m1_v[pl.ds(p1 * L, L)]
                ch = jnp.min(
                    jnp.where(m1c == gmax, p1 * L + iota16, big), axis=0
                )
                sch = s_v[pl.ds(ch * L, L)]
                idx = jnp.min(
                    jnp.where(sch == gmax, ch * L + iota16, big), axis=0
                )
                return gmax, idx

            def extract_f(ref, idx, lane):
                v = ref[pl.ds(idx - lane, L)]
                return jnp.max(jnp.where(iota16 == lane, v, NEG_INF), axis=0)

            def extract_i(ref, idx, lane):
                v = ref[pl.ds(idx - lane, L)]
                return jnp.max(jnp.where(iota16 == lane, v, 0), axis=0)

            def append(ref, val, qn, ln):
                v = ref[pl.ds(qn * L, L)]
                ref[pl.ds(qn * L, L)] = jnp.where(iota16 == ln, val, v)

            def cond(state):
                n, gmax, _idx = state
                return (n < MAX_OUT) & (gmax != NEG_INF)

            def body(state):
                n, gmax, idx = state
                lane = idx % L
                ch = idx // L
                bx1 = extract_f(x1_v, idx, lane)
                by1 = extract_f(y1_v, idx, lane)
                bx2 = extract_f(x2_v, idx, lane)
                by2 = extract_f(y2_v, idx, lane)
                barea = (bx2 - bx1) * (by2 - by1)

                sup = jnp.zeros((L,), jnp.int32)
                for q in range(SELPAD // L):
                    qx1 = sx1_v[pl.ds(q * L, L)]
                    qy1 = sy1_v[pl.ds(q * L, L)]
                    qx2 = sx2_v[pl.ds(q * L, L)]
                    qy2 = sy2_v[pl.ds(q * L, L)]
                    qar = sar_v[pl.ds(q * L, L)]
                    xx1 = jnp.maximum(bx1, qx1)
                    yy1 = jnp.maximum(by1, qy1)
                    xx2 = jnp.minimum(bx2, qx2)
                    yy2 = jnp.minimum(by2, qy2)
                    inter = jnp.maximum(xx2 - xx1, 0.0) * jnp.maximum(
                        yy2 - yy1, 0.0
                    )
                    iou = inter / (barea + qar - inter + 1e-9)
                    hit = (q * L + iota16 < n) & (iou > IOU_THR)
                    sup = sup | hit.astype(jnp.int32)
                keep = jnp.max(sup, axis=0) == 0

                sch = s_v[pl.ds(ch * L, L)]
                sch = jnp.where(iota16 == lane, NEG_INF, sch)
                s_v[pl.ds(ch * L, L)] = sch
                cmax = jnp.max(sch, axis=0)
                q1 = ch // L
                l1 = ch % L
                m1c = m1_v[pl.ds(q1 * L, L)]
                m1c = jnp.where(iota16 == l1, cmax, m1c)
                m1_v[pl.ds(q1 * L, L)] = m1c
                nm1 = jnp.max(m1c, axis=0)
                q2 = q1 // L
                l2 = q1 % L
                m2c = m2_v[pl.ds(q2 * L, L)]
                m2c = jnp.where(iota16 == l2, nm1, m2c)
                m2_v[pl.ds(q2 * L, L)] = m2c

                @pl.when(keep)
                def _():
                    bcls = extract_i(cl_v, idx, lane)
                    qn = n // L
                    ln = n % L
                    append(sx1_v, bx1, qn, ln)
                    append(sy1_v, by1, qn, ln)
                    append(sx2_v, bx2, qn, ln)
                    append(sy2_v, by2, qn, ln)
                    append(sar_v, barea, qn, ln)
                    append(oc_v, bcls, qn, ln)
                    append(os_v, gmax, qn, ln)

                n = n + keep.astype(jnp.int32)
                gmax, idx = global_argmax()
                return n, gmax, idx

            gmax0, idx0 = global_argmax()
            state = lax.while_loop(cond, body, (jnp.int32(0), gmax0, idx0))
            nfin = state[0]
            onv = on_v[pl.ds(0, L)]
            on_v[pl.ds(0, L)] = jnp.where(iota16 == 0, nfin, onv)

            pltpu.sync_copy(sx1_v, ox1_hbm.at[pl.ds(wid * SELPAD, SELPAD)])
            pltpu.sync_copy(sy1_v, oy1_hbm.at[pl.ds(wid * SELPAD, SELPAD)])
            pltpu.sync_copy(sx2_v, ox2_hbm.at[pl.ds(wid * SELPAD, SELPAD)])
            pltpu.sync_copy(sy2_v, oy2_hbm.at[pl.ds(wid * SELPAD, SELPAD)])
            pltpu.sync_copy(oc_v, oc_hbm.at[pl.ds(wid * SELPAD, SELPAD)])
            pltpu.sync_copy(os_v, os_hbm.at[pl.ds(wid * SELPAD, SELPAD)])
            pltpu.sync_copy(on_v, on_hbm.at[pl.ds(wid * L, L)])

    return k(f2, c2)


@jax.jit
def _run(lt, db):
    f, c = _prep(lt, db)
    f2 = f.reshape(4 * 5 * NPAD)
    c2 = c.reshape(4 * NPAD)
    ox1, oy1, ox2, oy2, oc, os_, on = _sc_nms(f2, c2)
    mask = jnp.arange(MAX_OUT)[None, :] < on.reshape(4, L)[:, :1]
    det_boxes = jnp.stack(
        [
            jnp.where(mask, ox1.reshape(4, SELPAD)[:, :MAX_OUT], 0.0),
            jnp.where(mask, oy1.reshape(4, SELPAD)[:, :MAX_OUT], 0.0),
            jnp.where(mask, ox2.reshape(4, SELPAD)[:, :MAX_OUT], 0.0),
            jnp.where(mask, oy2.reshape(4, SELPAD)[:, :MAX_OUT], 0.0),
        ],
        axis=-1,
    )
    return (
        det_boxes,
        oc.reshape(4, SELPAD)[:, :MAX_OUT],
        os_.reshape(4, SELPAD)[:, :MAX_OUT],
        on.reshape(4, L)[:, 0],
    )


def kernel(logits, default_boxes):
    b, n, c = logits.shape
    lt = jnp.transpose(logits, (0, 2, 1))
    lt = jnp.pad(lt, ((0, 0), (0, 0), (0, NPAD - n))).reshape(b, c, ROWS, LANES)
    db = jnp.pad(default_boxes.T, ((0, 0), (0, NPAD - n))).reshape(4, ROWS, LANES)
    return _run(lt, db)
